# baseline (speedup 1.0000x reference)
import functools

import jax
import jax.numpy as jnp
from jax import lax
from jax.experimental import pallas as pl
from jax.experimental.pallas import tpu as pltpu
from jax.experimental.pallas import tpu_sc as plsc

D_MODEL = 1024
_SCALE = float(1024.0 ** 0.5)

_NUM_CORES = 2
_NUM_SUBCORES = 16
_NW = _NUM_CORES * _NUM_SUBCORES

_LANES = 16
_GROUPS_PER_ROW = D_MODEL // _LANES

_C = 8
_NBUF = 4


def _body(idx_hbm, table_hbm, out_hbm, idx_v, ibufs, obufs,
          sg0, sg1, sg2, sg3, ss0, ss1, ss2, ss3, b_per_w):
    sem_g = (sg0, sg1, sg2, sg3)
    sem_s = (ss0, ss1, ss2, ss3)
    n_chunks = b_per_w // _C
    n_groups = n_chunks // _NBUF
    wid = lax.axis_index("s") * _NUM_CORES + lax.axis_index("c")
    base = wid * b_per_w

    pltpu.sync_copy(idx_hbm.at[pl.ds(base, b_per_w)], idx_v)

    def fire_gather(c, b):
        pltpu.async_copy(
            table_hbm.at[idx_v.at[pl.ds(c * _C, _C)]], ibufs.at[b], sem_g[b]
        )

    def wait_gather(b):
        pltpu.make_async_copy(
            table_hbm.at[idx_v.at[pl.ds(0, _C)]], ibufs.at[b], sem_g[b]
        ).wait()

    def fire_scatter(c, b):
        pltpu.async_copy(
            obufs.at[b], out_hbm.at[pl.ds(base + c * _C, _C)], sem_s[b]
        )

    def wait_scatter(b):
        pltpu.make_async_copy(
            obufs.at[b], out_hbm.at[pl.ds(base, _C)], sem_s[b]
        ).wait()

    def scale(b):
        @plsc.parallel_loop(0, _C)
        def _(r):
            for j in range(_GROUPS_PER_ROW):
                sl = pl.ds(j * _LANES, _LANES)
                obufs[b, r, sl] = ibufs[b, r, sl] * _SCALE

    fire_gather(0, 0)
    fire_gather(1, 1)

    for b in range(_NBUF):
        wait_gather(b)
        scale(b)
        fire_scatter(b, b)
        nb = (b + 2) % _NBUF
        if b >= 2:
            wait_scatter(nb)
        fire_gather(b + 2, nb)

    def group_body(t, _):
        c0 = t * _NBUF
        for b in range(_NBUF):
            c = c0 + b
            wait_gather(b)
            scale(b)
            fire_scatter(c, b)
            nb = (b + 2) % _NBUF
            wait_scatter(nb)
            fire_gather(c + 2, nb)
        return 0

    lax.fori_loop(1, n_groups - 1, group_body, 0, unroll=False)

    c0 = n_chunks - _NBUF
    for b in range(_NBUF):
        c = c0 + b
        wait_gather(b)
        scale(b)
        fire_scatter(c, b)
        nb = (b + 2) % _NBUF
        wait_scatter(nb)
        if b < 2:
            fire_gather(c + 2, nb)

    wait_scatter(2)
    wait_scatter(3)


def kernel(x, table):
    b, s = x.shape
    n = b * s
    idx = x.reshape(n).astype(jnp.int32)
    b_per_w = n // _NW

    mesh = plsc.VectorSubcoreMesh(
        core_axis_name="c", subcore_axis_name="s"
    )
    run = pl.kernel(
        functools.partial(_body, b_per_w=b_per_w),
        out_type=jax.ShapeDtypeStruct((n, D_MODEL), jnp.float32),
        mesh=mesh,
        scratch_types=[
            pltpu.VMEM((b_per_w,), jnp.int32),
            pltpu.VMEM((_NBUF, _C, D_MODEL), jnp.float32),
            pltpu.VMEM((_NBUF, _C, D_MODEL), jnp.float32),
        ] + [pltpu.SemaphoreType.DMA] * (2 * _NBUF),
    )
    out = run(idx, table)
    return out.reshape(b, s, D_MODEL)


# speedup vs baseline: 1.0601x; 1.0601x over previous
import functools

import jax
import jax.numpy as jnp
from jax import lax
from jax.experimental import pallas as pl
from jax.experimental.pallas import tpu as pltpu
from jax.experimental.pallas import tpu_sc as plsc

D_MODEL = 1024
_SCALE = float(1024.0 ** 0.5)

_NUM_CORES = 2
_NUM_SUBCORES = 16
_NW = _NUM_CORES * _NUM_SUBCORES

_LANES = 16
_GROUPS_PER_ROW = D_MODEL // _LANES

_C = 16
_NBUF = 3


def _body(idx_hbm, table_hbm, out_hbm, idx_v, ibufs, obufs,
          sg0, sg1, sg2, ss0, ss1, ss2, b_per_w):
    sem_g = (sg0, sg1, sg2)
    sem_s = (ss0, ss1, ss2)
    n_chunks = b_per_w // _C
    wid = lax.axis_index("s") * _NUM_CORES + lax.axis_index("c")
    base = wid * b_per_w

    pltpu.sync_copy(idx_hbm.at[pl.ds(base, b_per_w)], idx_v)

    def fire_gather(c, b):
        pltpu.async_copy(
            table_hbm.at[idx_v.at[pl.ds(c * _C, _C)]], ibufs.at[b], sem_g[b]
        )

    def wait_gather(b):
        pltpu.make_async_copy(
            table_hbm.at[idx_v.at[pl.ds(0, _C)]], ibufs.at[b], sem_g[b]
        ).wait()

    def fire_scatter(c, b):
        pltpu.async_copy(
            obufs.at[b], out_hbm.at[pl.ds(base + c * _C, _C)], sem_s[b]
        )

    def wait_scatter(b):
        pltpu.make_async_copy(
            obufs.at[b], out_hbm.at[pl.ds(base, _C)], sem_s[b]
        ).wait()

    def scale(b):
        @plsc.parallel_loop(0, _C)
        def _(r):
            for j in range(_GROUPS_PER_ROW):
                sl = pl.ds(j * _LANES, _LANES)
                obufs[b, r, sl] = ibufs[b, r, sl] * _SCALE

    fire_gather(0, 0)
    fire_gather(1, 1)
    for c in range(_NBUF):
        b = c % _NBUF
        wait_gather(b)
        scale(b)
        fire_scatter(c, b)
        fire_gather(c + 2, (c + 2) % _NBUF)

    def group_body(t, _):
        c0 = t * _NBUF
        for b in range(_NBUF):
            c = c0 + b
            wait_gather(b)
            wait_scatter(b)
            scale(b)
            fire_scatter(c, b)
            fire_gather(c + 2, (b + 2) % _NBUF)
        return 0

    lax.fori_loop(1, (n_chunks - 2) // _NBUF, group_body, 0, unroll=False)

    for c in range(n_chunks - 2, n_chunks):
        b = c % _NBUF
        wait_gather(b)
        wait_scatter(b)
        scale(b)
        fire_scatter(c, b)

    for c in range(n_chunks - _NBUF, n_chunks):
        wait_scatter(c % _NBUF)


def kernel(x, table):
    b, s = x.shape
    n = b * s
    idx = x.reshape(n).astype(jnp.int32)
    b_per_w = n // _NW

    mesh = plsc.VectorSubcoreMesh(
        core_axis_name="c", subcore_axis_name="s"
    )
    run = pl.kernel(
        functools.partial(_body, b_per_w=b_per_w),
        out_type=jax.ShapeDtypeStruct((n, D_MODEL), jnp.float32),
        mesh=mesh,
        scratch_types=[
            pltpu.VMEM((b_per_w,), jnp.int32),
            pltpu.VMEM((_NBUF, _C, D_MODEL), jnp.float32),
            pltpu.VMEM((_NBUF, _C, D_MODEL), jnp.float32),
        ] + [pltpu.SemaphoreType.DMA] * (2 * _NBUF),
    )
    out = run(idx, table)
    return out.reshape(b, s, D_MODEL)


# speedup vs baseline: 1.1046x; 1.0420x over previous
import functools

import jax
import jax.numpy as jnp
from jax import lax
from jax.experimental import pallas as pl
from jax.experimental.pallas import tpu as pltpu
from jax.experimental.pallas import tpu_sc as plsc

D_MODEL = 1024
_SCALE = float(1024.0 ** 0.5)

_NUM_CORES = 2
_NUM_SUBCORES = 16
_NW = _NUM_CORES * _NUM_SUBCORES

_LANES = 16
_GROUPS_PER_ROW = D_MODEL // _LANES

_C = 16
_NBUF = 4


def _body(idx_hbm, table_hbm, out_hbm, idx_v, bufs, sg0, sg1, sg2, sg3,
          ss0, ss1, ss2, ss3, b_per_w):
    sem_g = (sg0, sg1, sg2, sg3)
    sem_s = (ss0, ss1, ss2, ss3)
    n_chunks = b_per_w // _C
    n_groups = n_chunks // _NBUF
    wid = lax.axis_index("s") * _NUM_CORES + lax.axis_index("c")
    base = wid * b_per_w

    pltpu.sync_copy(idx_hbm.at[pl.ds(base, b_per_w)], idx_v)

    def fire_gather(c, b):
        pltpu.async_copy(
            table_hbm.at[idx_v.at[pl.ds(c * _C, _C)]], bufs.at[b], sem_g[b]
        )

    def wait_gather(b):
        pltpu.make_async_copy(
            table_hbm.at[idx_v.at[pl.ds(0, _C)]], bufs.at[b], sem_g[b]
        ).wait()

    def fire_scatter(c, b):
        pltpu.async_copy(
            bufs.at[b], out_hbm.at[pl.ds(base + c * _C, _C)], sem_s[b]
        )

    def wait_scatter(b):
        pltpu.make_async_copy(
            bufs.at[b], out_hbm.at[pl.ds(base, _C)], sem_s[b]
        ).wait()

    def scale(b):
        @plsc.parallel_loop(0, _C)
        def _(r):
            for j in range(_GROUPS_PER_ROW):
                sl = pl.ds(j * _LANES, _LANES)
                bufs[b, r, sl] = bufs[b, r, sl] * _SCALE

    fire_gather(0, 0)
    fire_gather(1, 1)

    for b in range(_NBUF):
        wait_gather(b)
        nb = (b + 2) % _NBUF
        if b >= 2:
            wait_scatter(nb)
        fire_gather(b + 2, nb)
        scale(b)
        fire_scatter(b, b)

    def group_body(t, _):
        c0 = t * _NBUF
        for b in range(_NBUF):
            c = c0 + b
            wait_gather(b)
            nb = (b + 2) % _NBUF
            wait_scatter(nb)
            fire_gather(c + 2, nb)
            scale(b)
            fire_scatter(c, b)
        return 0

    lax.fori_loop(1, n_groups - 1, group_body, 0, unroll=False)

    c0 = n_chunks - _NBUF
    for b in range(_NBUF):
        c = c0 + b
        wait_gather(b)
        nb = (b + 2) % _NBUF
        wait_scatter(nb)
        if b < 2:
            fire_gather(c + 2, nb)
        scale(b)
        fire_scatter(c, b)

    wait_scatter(2)
    wait_scatter(3)


def kernel(x, table):
    b, s = x.shape
    n = b * s
    idx = x.reshape(n).astype(jnp.int32)
    b_per_w = n // _NW

    mesh = plsc.VectorSubcoreMesh(
        core_axis_name="c", subcore_axis_name="s"
    )
    run = pl.kernel(
        functools.partial(_body, b_per_w=b_per_w),
        out_type=jax.ShapeDtypeStruct((n, D_MODEL), jnp.float32),
        mesh=mesh,
        scratch_types=[
            pltpu.VMEM((b_per_w,), jnp.int32),
            pltpu.VMEM((_NBUF, _C, D_MODEL), jnp.float32),
        ] + [pltpu.SemaphoreType.DMA] * (2 * _NBUF),
    )
    out = run(idx, table)
    return out.reshape(b, s, D_MODEL)


# speedup vs baseline: 1.1420x; 1.0338x over previous
import functools

import jax
import jax.numpy as jnp
from jax import lax
from jax.experimental import pallas as pl
from jax.experimental.pallas import tpu as pltpu
from jax.experimental.pallas import tpu_sc as plsc

D_MODEL = 1024
_SCALE = float(1024.0 ** 0.5)

_NUM_CORES = 2
_NUM_SUBCORES = 16
_NW = _NUM_CORES * _NUM_SUBCORES

_LANES = 16
_GROUPS_PER_ROW = D_MODEL // _LANES

_C = 32
_NBUF = 3


def _body(idx_hbm, table_hbm, out_hbm, idx_v, bufs, sg0, sg1, sg2,
          ss0, ss1, ss2, b_per_w):
    sem_g = (sg0, sg1, sg2)
    sem_s = (ss0, ss1, ss2)
    n_chunks = b_per_w // _C
    wid = lax.axis_index("s") * _NUM_CORES + lax.axis_index("c")
    base = wid * b_per_w

    pltpu.sync_copy(idx_hbm.at[pl.ds(base, b_per_w)], idx_v)

    def fire_gather(c, b):
        pltpu.async_copy(
            table_hbm.at[idx_v.at[pl.ds(c * _C, _C)]], bufs.at[b], sem_g[b]
        )

    def wait_gather(b):
        pltpu.make_async_copy(
            table_hbm.at[idx_v.at[pl.ds(0, _C)]], bufs.at[b], sem_g[b]
        ).wait()

    def fire_scatter(c, b):
        pltpu.async_copy(
            bufs.at[b], out_hbm.at[pl.ds(base + c * _C, _C)], sem_s[b]
        )

    def wait_scatter(b):
        pltpu.make_async_copy(
            bufs.at[b], out_hbm.at[pl.ds(base, _C)], sem_s[b]
        ).wait()

    def scale(b):
        @plsc.parallel_loop(0, _C)
        def _(r):
            for j in range(_GROUPS_PER_ROW):
                sl = pl.ds(j * _LANES, _LANES)
                bufs[b, r, sl] = bufs[b, r, sl] * _SCALE

    fire_gather(0, 0)

    for c in range(_NBUF):
        b = c % _NBUF
        wait_gather(b)
        nb = (b + 1) % _NBUF
        if c >= 2:
            wait_scatter(nb)
        fire_gather(c + 1, nb)
        scale(b)
        fire_scatter(c, b)

    def group_body(t, _):
        c0 = t * _NBUF
        for b in range(_NBUF):
            c = c0 + b
            wait_gather(b)
            nb = (b + 1) % _NBUF
            wait_scatter(nb)
            fire_gather(c + 1, nb)
            scale(b)
            fire_scatter(c, b)
        return 0

    lax.fori_loop(1, (n_chunks - 1) // _NBUF, group_body, 0, unroll=False)

    c = n_chunks - 1
    b = c % _NBUF
    wait_gather(b)
    wait_scatter((b + 1) % _NBUF)
    scale(b)
    fire_scatter(c, b)

    wait_scatter((b + 2) % _NBUF)
    wait_scatter(b)


def kernel(x, table):
    b, s = x.shape
    n = b * s
    idx = x.reshape(n).astype(jnp.int32)
    b_per_w = n // _NW

    mesh = plsc.VectorSubcoreMesh(
        core_axis_name="c", subcore_axis_name="s"
    )
    run = pl.kernel(
        functools.partial(_body, b_per_w=b_per_w),
        out_type=jax.ShapeDtypeStruct((n, D_MODEL), jnp.float32),
        mesh=mesh,
        scratch_types=[
            pltpu.VMEM((b_per_w,), jnp.int32),
            pltpu.VMEM((_NBUF, _C, D_MODEL), jnp.float32),
        ] + [pltpu.SemaphoreType.DMA] * (2 * _NBUF),
    )
    out = run(idx, table)
    return out.reshape(b, s, D_MODEL)
